# baseline (speedup 1.0000x reference)
import functools

import jax
import jax.numpy as jnp
from jax import lax
from jax.experimental import pallas as pl
from jax.experimental.pallas import tpu as pltpu
from jax.experimental.pallas import tpu_sc as plsc

_LANES = 16
_CH = 128


def _sc_gather(img_idx, dR_flat, dT_flat, n_images):
    B = img_idx.shape[0]
    info = plsc.get_sparse_core_info()
    nc, ns = info.num_cores, info.num_subcores
    nw = nc * ns
    bpw = B // nw
    assert B % (nw * _CH) == 0
    rpw = bpw // _CH
    n_ch = 3 * rpw

    mesh = plsc.VectorSubcoreMesh(core_axis_name="c", subcore_axis_name="s")

    @functools.partial(
        pl.kernel,
        mesh=mesh,
        out_type=jax.ShapeDtypeStruct((6, nw, bpw), jnp.float32),
        scratch_types=[
            pltpu.VMEM((bpw,), jnp.int32),
            pltpu.VMEM((2 * bpw,), jnp.int32),
            pltpu.VMEM((3 * bpw,), jnp.float32),
            pltpu.VMEM((3 * bpw,), jnp.float32),
        ]
        + [pltpu.SemaphoreType.DMA] * 4,
    )
    def k(idx_hbm, dR_hbm, dT_hbm, out_hbm, idx_v, ci_v, vr_v, vt_v, *sems):
        wid = lax.axis_index("s") * nc + lax.axis_index("c")
        base = wid * bpw
        pltpu.sync_copy(idx_hbm.at[pl.ds(base, bpw)], idx_v)
        c0r = pltpu.async_copy(dR_hbm.at[idx_v], vr_v.at[pl.ds(0, bpw)], sems[0])
        c0t = pltpu.async_copy(dT_hbm.at[idx_v], vt_v.at[pl.ds(0, bpw)], sems[1])
        for c in (1, 2):
            for i in range(bpw // _LANES):
                v = idx_v[pl.ds(i * _LANES, _LANES)]
                ci_v[pl.ds((c - 1) * bpw + i * _LANES, _LANES)] = v + c * n_images
        c12r = pltpu.async_copy(dR_hbm.at[ci_v], vr_v.at[pl.ds(bpw, 2 * bpw)], sems[2])
        c12t = pltpu.async_copy(dT_hbm.at[ci_v], vt_v.at[pl.ds(bpw, 2 * bpw)], sems[3])
        c0r.wait()
        pltpu.sync_copy(vr_v.at[pl.ds(0, bpw)], out_hbm.at[0, wid])
        c0t.wait()
        pltpu.sync_copy(vt_v.at[pl.ds(0, bpw)], out_hbm.at[3, wid])
        c12r.wait()
        for c in (1, 2):
            pltpu.sync_copy(vr_v.at[pl.ds(c * bpw, bpw)], out_hbm.at[c, wid])
        c12t.wait()
        for c in (1, 2):
            pltpu.sync_copy(vt_v.at[pl.ds(c * bpw, bpw)], out_hbm.at[3 + c, wid])

    return k(img_idx, dR_flat, dT_flat)


def _tc_body(g_ref, p_ref, o_ref):
    x = g_ref[0]
    y = g_ref[1]
    z = g_ref[2]
    xx = x * x
    yy = y * y
    zz = z * z
    s = xx + yy + zz
    n = jnp.sqrt(s) + 1e-7
    a = jnp.sin(n) / n
    b = (1.0 - jnp.cos(n)) / (n * n)
    ax = a * x
    ay = a * y
    az = a * z
    bxy = b * x * y
    bxz = b * x * z
    byz = b * y * z
    r0 = (1.0 + b * (xx - s), bxy - az, bxz + ay)
    r1 = (bxy + az, 1.0 + b * (yy - s), byz - ax)
    r2 = (bxz - ay, byz + ax, 1.0 + b * (zz - s))
    p = [p_ref[k] for k in range(12)]
    for i, row in enumerate((r0, r1, r2)):
        for j in range(3):
            o_ref[4 * i + j] = row[0] * p[j] + row[1] * p[4 + j] + row[2] * p[8 + j]
        o_ref[4 * i + 3] = p[4 * i + 3] + g_ref[3 + i]


def kernel(img_idx, poses, dR_param, dT_param):
    B = img_idx.shape[0]
    assert B % _CH == 0
    rows = B // _CH
    g = _sc_gather(
        img_idx,
        dR_param.T.reshape(-1),
        dT_param.T.reshape(-1),
        dR_param.shape[0],
    )
    g = g.reshape(6, rows, _CH)
    pose_t = poses.reshape(B, 12).T.reshape(12, rows, _CH)
    out_t = pl.pallas_call(
        _tc_body,
        out_shape=jax.ShapeDtypeStruct((12, rows, _CH), jnp.float32),
    )(g, pose_t)
    return out_t.reshape(12, B).T.reshape(B, 3, 4)


# speedup vs baseline: 1.1115x; 1.1115x over previous
import functools

import jax
import jax.numpy as jnp
from jax import lax
from jax.experimental import pallas as pl
from jax.experimental.pallas import tpu as pltpu
from jax.experimental.pallas import tpu_sc as plsc

_LANES = 16
_CH = 128


def _sc_gather(img_idx, dR_flat, dT_flat, n_images):
    B = img_idx.shape[0]
    info = plsc.get_sparse_core_info()
    nc, ns = info.num_cores, info.num_subcores
    nw = nc * ns
    bpw = B // nw
    assert B % (nw * _CH) == 0
    rpw = bpw // _CH

    mesh = plsc.VectorSubcoreMesh(core_axis_name="c", subcore_axis_name="s")

    @functools.partial(
        pl.kernel,
        mesh=mesh,
        out_type=jax.ShapeDtypeStruct((6, nw, rpw, _CH), jnp.float32),
        scratch_types=[
            pltpu.VMEM((bpw,), jnp.int32),
            pltpu.VMEM((2, bpw), jnp.int32),
            pltpu.VMEM((6, rpw, _CH), jnp.float32),
        ]
        + [pltpu.SemaphoreType.DMA] * 6,
    )
    def k(idx_hbm, dR_hbm, dT_hbm, out_hbm, idx_v, ci_v, val_v, *sems):
        wid = lax.axis_index("s") * nc + lax.axis_index("c")
        base = wid * bpw
        with jax.named_scope("idxload"):
            pltpu.sync_copy(idx_hbm.at[pl.ds(base, bpw)], idx_v)
        copies = [[] for _ in range(6)]
        with jax.named_scope("fire"):
            for r in range(rpw):
                isl = idx_v.at[pl.ds(r * _CH, _CH)]
                copies[0].append(
                    pltpu.async_copy(dR_hbm.at[isl], val_v.at[0, r], sems[0])
                )
                copies[3].append(
                    pltpu.async_copy(dT_hbm.at[isl], val_v.at[3, r], sems[3])
                )
            for c in (1, 2):
                def body(i, carry, c=c):
                    pos = i * _LANES
                    ci_v[c - 1, pl.ds(pos, _LANES)] = (
                        idx_v[pl.ds(pos, _LANES)] + c * n_images
                    )
                    return carry
                lax.fori_loop(0, bpw // _LANES, body, 0)
                for r in range(rpw):
                    isl = ci_v.at[c - 1, pl.ds(r * _CH, _CH)]
                    copies[c].append(
                        pltpu.async_copy(dR_hbm.at[isl], val_v.at[c, r], sems[c])
                    )
                    copies[3 + c].append(
                        pltpu.async_copy(
                            dT_hbm.at[isl], val_v.at[3 + c, r], sems[3 + c]
                        )
                    )
        with jax.named_scope("drain"):
            for c in range(6):
                for cp in copies[c]:
                    cp.wait()
                pltpu.sync_copy(val_v.at[c], out_hbm.at[c, wid])

    return k(img_idx, dR_flat, dT_flat)


def _tc_body(g_ref, p_ref, o_ref):
    x = g_ref[0]
    y = g_ref[1]
    z = g_ref[2]
    xx = x * x
    yy = y * y
    zz = z * z
    s = xx + yy + zz
    n = jnp.sqrt(s) + 1e-7
    a = jnp.sin(n) / n
    b = (1.0 - jnp.cos(n)) / (n * n)
    ax = a * x
    ay = a * y
    az = a * z
    bxy = b * x * y
    bxz = b * x * z
    byz = b * y * z
    r0 = (1.0 + b * (xx - s), bxy - az, bxz + ay)
    r1 = (bxy + az, 1.0 + b * (yy - s), byz - ax)
    r2 = (bxz - ay, byz + ax, 1.0 + b * (zz - s))
    p = [p_ref[k] for k in range(12)]
    for i, row in enumerate((r0, r1, r2)):
        for j in range(3):
            o_ref[i, :, j, :] = row[0] * p[j] + row[1] * p[4 + j] + row[2] * p[8 + j]
        o_ref[i, :, 3, :] = p[4 * i + 3] + g_ref[3 + i]


def kernel(img_idx, poses, dR_param, dT_param):
    B = img_idx.shape[0]
    assert B % _CH == 0
    rows = B // _CH
    g = _sc_gather(
        img_idx,
        dR_param.T.reshape(-1),
        dT_param.T.reshape(-1),
        dR_param.shape[0],
    )
    g = g.reshape(6, rows, _CH)
    pose_t = poses.reshape(B, 12).T.reshape(12, rows, _CH)
    grid = 4
    rb = rows // grid
    out_t = pl.pallas_call(
        _tc_body,
        grid=(grid,),
        in_specs=[
            pl.BlockSpec((6, rb, _CH), lambda i: (0, i, 0)),
            pl.BlockSpec((12, rb, _CH), lambda i: (0, i, 0)),
        ],
        out_specs=pl.BlockSpec((3, rb, 4, _CH), lambda i: (0, i, 0, 0)),
        out_shape=jax.ShapeDtypeStruct((3, rows, 4, _CH), jnp.float32),
    )(g, pose_t)
    return out_t.transpose(1, 3, 0, 2).reshape(B, 3, 4)


# speedup vs baseline: 1.1453x; 1.0305x over previous
import functools

import jax
import jax.numpy as jnp
from jax import lax
from jax.experimental import pallas as pl
from jax.experimental.pallas import tpu as pltpu
from jax.experimental.pallas import tpu_sc as plsc

_LANES = 16
_CH = 128


def _sc_gather(img_idx, dR_flat, dT_flat, n_images):
    B = img_idx.shape[0]
    info = plsc.get_sparse_core_info()
    nc, ns = info.num_cores, info.num_subcores
    nw = nc * ns
    bpw = B // nw
    assert B % (nw * _CH) == 0
    rpw = bpw // _CH

    mesh = plsc.VectorSubcoreMesh(core_axis_name="c", subcore_axis_name="s")

    @functools.partial(
        pl.kernel,
        mesh=mesh,
        out_type=jax.ShapeDtypeStruct((6, nw, rpw, _CH), jnp.float32),
        scratch_types=[
            pltpu.VMEM((bpw,), jnp.int32),
            pltpu.VMEM((2, bpw), jnp.int32),
            pltpu.VMEM((6, rpw, _CH), jnp.float32),
        ]
        + [pltpu.SemaphoreType.DMA] * 6,
    )
    def k(idx_hbm, dR_hbm, dT_hbm, out_hbm, idx_v, ci_v, val_v, *sems):
        wid = lax.axis_index("s") * nc + lax.axis_index("c")
        base = wid * bpw
        pltpu.sync_copy(idx_hbm.at[pl.ds(base, bpw)], idx_v)
        copies = [[] for _ in range(6)]
        for r in range(rpw):
            isl = idx_v.at[pl.ds(r * _CH, _CH)]
            copies[0].append(
                pltpu.async_copy(dR_hbm.at[isl], val_v.at[0, r], sems[0])
            )
            copies[3].append(
                pltpu.async_copy(dT_hbm.at[isl], val_v.at[3, r], sems[3])
            )
        for c in (1, 2):
            def body(i, carry, c=c):
                pos = i * _LANES
                ci_v[c - 1, pl.ds(pos, _LANES)] = (
                    idx_v[pl.ds(pos, _LANES)] + c * n_images
                )
                return carry
            lax.fori_loop(0, bpw // _LANES, body, 0)
            for r in range(rpw):
                isl = ci_v.at[c - 1, pl.ds(r * _CH, _CH)]
                copies[c].append(
                    pltpu.async_copy(dR_hbm.at[isl], val_v.at[c, r], sems[c])
                )
                copies[3 + c].append(
                    pltpu.async_copy(
                        dT_hbm.at[isl], val_v.at[3 + c, r], sems[3 + c]
                    )
                )
        for c in range(6):
            for cp in copies[c]:
                cp.wait()
            pltpu.sync_copy(val_v.at[c], out_hbm.at[c, wid])

    return k(img_idx, dR_flat, dT_flat)


def _tc_body(g_ref, p_ref, o_ref):
    x = g_ref[0]
    y = g_ref[1]
    z = g_ref[2]
    xx = x * x
    yy = y * y
    zz = z * z
    s = xx + yy + zz
    n = jnp.sqrt(s) + 1e-7
    a = jnp.sin(n) / n
    b = (1.0 - jnp.cos(n)) / (n * n)
    ax = a * x
    ay = a * y
    az = a * z
    bxy = b * x * y
    bxz = b * x * z
    byz = b * y * z
    r0 = (1.0 + b * (xx - s), bxy - az, bxz + ay)
    r1 = (bxy + az, 1.0 + b * (yy - s), byz - ax)
    r2 = (bxz - ay, byz + ax, 1.0 + b * (zz - s))
    p = [p_ref[k] for k in range(12)]
    for i, row in enumerate((r0, r1, r2)):
        for j in range(3):
            o_ref[i, :, j, :] = row[0] * p[j] + row[1] * p[4 + j] + row[2] * p[8 + j]
        o_ref[i, :, 3, :] = p[4 * i + 3] + g_ref[3 + i]


def kernel(img_idx, poses, dR_param, dT_param):
    B = img_idx.shape[0]
    assert B % _CH == 0
    rows = B // _CH
    g = _sc_gather(
        img_idx,
        dR_param.T.reshape(-1),
        dT_param.T.reshape(-1),
        dR_param.shape[0],
    )
    g = g.reshape(6, rows, _CH)
    pose_t = poses.reshape(B, 12).T.reshape(12, rows, _CH)
    out_t = pl.pallas_call(
        _tc_body,
        out_shape=jax.ShapeDtypeStruct((3, rows, 4, _CH), jnp.float32),
    )(g, pose_t)
    return out_t.transpose(1, 3, 0, 2).reshape(B, 3, 4)
